# baseline (speedup 1.0000x reference)
import functools

import jax
import jax.numpy as jnp
from jax import lax
from jax.experimental import pallas as pl
from jax.experimental.pallas import tpu as pltpu
from jax.experimental.pallas import tpu_sc as plsc

_BATCH = 16384
_EMB = 64
_NROWS = 100000

_info = plsc.get_sparse_core_info()
_NC, _NS = _info.num_cores, _info.num_subcores
_NW = _NC * _NS
_BPW = _BATCH // _NW


def _tpose_body(pt_ref, out_ref):
  out_ref[...] = pt_ref[...].T


_TP_BB = 16384
_TP_GRID = -(-_NROWS // _TP_BB)


def _tpose_call(pt):
  return pl.pallas_call(
      _tpose_body,
      grid=(_TP_GRID,),
      in_specs=[pl.BlockSpec((_EMB, _TP_BB), lambda i: (0, i))],
      out_specs=pl.BlockSpec((_TP_BB, _EMB), lambda i: (i, 0)),
      out_shape=jax.ShapeDtypeStruct((_NROWS, _EMB), jnp.float32),
  )(pt)


def _sc_gather_body(idx_hbm, emb_hbm, out_hbm, idx_v, rows_v, sem):
  wid = lax.axis_index("s") * _NC + lax.axis_index("c")
  base = wid * _BPW
  pltpu.sync_copy(idx_hbm.at[pl.ds(base, _BPW)], idx_v)

  def lp(c, _):
    ci = idx_v[pl.ds(c * 16, 16)]
    for k in range(16):
      pltpu.async_copy(emb_hbm.at[pl.ds(ci[k], 1)],
                       rows_v.at[pl.ds(c * 16 + k, 1)], sem)
    return 0

  lax.fori_loop(0, _BPW // 16, lp, 0)
  pltpu.make_async_copy(out_hbm.at[pl.ds(base, _BPW)], rows_v, sem).wait()
  pltpu.sync_copy(rows_v, out_hbm.at[pl.ds(base, _BPW)])


_sc_gather = pl.kernel(
    _sc_gather_body,
    out_type=jax.ShapeDtypeStruct((_BATCH, _EMB), jnp.float32),
    mesh=plsc.VectorSubcoreMesh(core_axis_name="c", subcore_axis_name="s"),
    scratch_types=[
        pltpu.VMEM((_BPW,), jnp.int32),
        pltpu.VMEM((_BPW, _EMB), jnp.float32),
        pltpu.SemaphoreType.DMA,
    ],
)


def _mlp_body(u_ref, v_ref, w1u_ref, w1v_ref, b1_ref, w2_ref, b2_ref,
              w3_ref, b3_ref, out_ref):
  x1 = (jnp.dot(u_ref[...], w1u_ref[...], preferred_element_type=jnp.float32)
        + jnp.dot(v_ref[...], w1v_ref[...], preferred_element_type=jnp.float32)
        + b1_ref[...])
  h1 = jnp.maximum(x1, 0.0)
  h2 = jnp.maximum(
      jnp.dot(h1, w2_ref[...], preferred_element_type=jnp.float32)
      + b2_ref[...], 0.0)
  out_ref[...] = (
      jnp.dot(w3_ref[...].T, h2.T, preferred_element_type=jnp.float32)
      + b3_ref[0, 0])


_MLP_BB = 8192


def _mlp_call(u, v, w1u, w1v, b1, w2t, b2, w3t, b3):
  grid = (_BATCH // _MLP_BB,)
  return pl.pallas_call(
      _mlp_body,
      grid=grid,
      in_specs=[
          pl.BlockSpec((_MLP_BB, _EMB), lambda i: (i, 0)),
          pl.BlockSpec((_MLP_BB, _EMB), lambda i: (i, 0)),
          pl.BlockSpec(w1u.shape, lambda i: (0, 0)),
          pl.BlockSpec(w1v.shape, lambda i: (0, 0)),
          pl.BlockSpec(b1.shape, lambda i: (0, 0)),
          pl.BlockSpec(w2t.shape, lambda i: (0, 0)),
          pl.BlockSpec(b2.shape, lambda i: (0, 0)),
          pl.BlockSpec(w3t.shape, lambda i: (0, 0)),
          pl.BlockSpec(b3.shape, lambda i: (0, 0)),
      ],
      out_specs=pl.BlockSpec((1, _MLP_BB), lambda i: (0, i)),
      out_shape=jax.ShapeDtypeStruct((1, _BATCH), jnp.float32),
  )(u, v, w1u, w1v, b1, w2t, b2, w3t, b3)


@jax.jit
def kernel(users, items, user_emb, item_emb, W1, b1, W2, b2, W3, b3):
  uemb_rm = _tpose_call(user_emb.T)
  u = _sc_gather(users, uemb_rm)
  iemb_rm = _tpose_call(item_emb.T)
  v = _sc_gather(items, iemb_rm)
  w1t = W1.T
  w1u = w1t[:_EMB]
  w1v = w1t[_EMB:]
  out = _mlp_call(u, v, w1u, w1v, b1.reshape(1, -1), W2.T,
                  b2.reshape(1, -1), W3.T, b3.reshape(1, 1))
  return out.reshape(_BATCH)


# speedup vs baseline: 1.0058x; 1.0058x over previous
import functools

import jax
import jax.numpy as jnp
from jax import lax
from jax.experimental import pallas as pl
from jax.experimental.pallas import tpu as pltpu
from jax.experimental.pallas import tpu_sc as plsc

_BATCH = 16384
_EMB = 64
_NROWS = 100000

_info = plsc.get_sparse_core_info()
_NC, _NS = _info.num_cores, _info.num_subcores
_NW = _NC * _NS
_BPW = _BATCH // _NW


def _tpose_body(pt_ref, out_ref):
  out_ref[...] = pt_ref[...].T


_TP_BB = 16384
_TP_GRID = -(-_NROWS // _TP_BB)


def _tpose_call(pt):
  return pl.pallas_call(
      _tpose_body,
      grid=(_TP_GRID,),
      in_specs=[pl.BlockSpec((_EMB, _TP_BB), lambda i: (0, i))],
      out_specs=pl.BlockSpec((_TP_BB, _EMB), lambda i: (i, 0)),
      out_shape=jax.ShapeDtypeStruct((_NROWS, _EMB), jnp.float32),
  )(pt)


def _sc_gather_body(idx_hbm, emb_hbm, out_hbm, idx_v, rows_v, sem):
  wid = lax.axis_index("s") * _NC + lax.axis_index("c")
  base = wid * _BPW
  pltpu.sync_copy(idx_hbm.at[pl.ds(base, _BPW)], idx_v)

  def lp(c, _):
    ci = idx_v[pl.ds(c * 16, 16)]
    for k in range(16):
      pltpu.async_copy(emb_hbm.at[pl.ds(ci[k], 1)],
                       rows_v.at[pl.ds(c * 16 + k, 1)], sem)
    return 0

  lax.fori_loop(0, _BPW // 16, lp, 0)
  pltpu.make_async_copy(out_hbm.at[pl.ds(base, _BPW)], rows_v, sem).wait()
  pltpu.sync_copy(rows_v, out_hbm.at[pl.ds(base, _BPW)])


_sc_gather = pl.kernel(
    _sc_gather_body,
    out_type=jax.ShapeDtypeStruct((_BATCH, _EMB), jnp.float32),
    mesh=plsc.VectorSubcoreMesh(core_axis_name="c", subcore_axis_name="s"),
    scratch_types=[
        pltpu.VMEM((_BPW,), jnp.int32),
        pltpu.VMEM((_BPW, _EMB), jnp.float32),
        pltpu.SemaphoreType.DMA,
    ],
)


def _mlp_body(u_ref, v_ref, w1u_ref, w1v_ref, b1_ref, w2_ref, b2_ref,
              w3_ref, b3_ref, out_ref):
  x1 = (jnp.dot(u_ref[...], w1u_ref[...], preferred_element_type=jnp.float32)
        + jnp.dot(v_ref[...], w1v_ref[...], preferred_element_type=jnp.float32)
        + b1_ref[...])
  h1 = jnp.maximum(x1, 0.0)
  h2 = jnp.maximum(
      jnp.dot(h1, w2_ref[...], preferred_element_type=jnp.float32)
      + b2_ref[...], 0.0)
  out_ref[...] = (
      jnp.dot(w3_ref[...].T, h2.T, preferred_element_type=jnp.float32)
      + b3_ref[0, 0])


_MLP_BB = 4096


def _mlp_call(u, v, w1u, w1v, b1, w2t, b2, w3t, b3):
  grid = (_BATCH // _MLP_BB,)
  return pl.pallas_call(
      _mlp_body,
      grid=grid,
      in_specs=[
          pl.BlockSpec((_MLP_BB, _EMB), lambda i: (i, 0)),
          pl.BlockSpec((_MLP_BB, _EMB), lambda i: (i, 0)),
          pl.BlockSpec(w1u.shape, lambda i: (0, 0)),
          pl.BlockSpec(w1v.shape, lambda i: (0, 0)),
          pl.BlockSpec(b1.shape, lambda i: (0, 0)),
          pl.BlockSpec(w2t.shape, lambda i: (0, 0)),
          pl.BlockSpec(b2.shape, lambda i: (0, 0)),
          pl.BlockSpec(w3t.shape, lambda i: (0, 0)),
          pl.BlockSpec(b3.shape, lambda i: (0, 0)),
      ],
      out_specs=pl.BlockSpec((1, _MLP_BB), lambda i: (0, i)),
      out_shape=jax.ShapeDtypeStruct((1, _BATCH), jnp.float32),
  )(u, v, w1u, w1v, b1, w2t, b2, w3t, b3)


@jax.jit
def kernel(users, items, user_emb, item_emb, W1, b1, W2, b2, W3, b3):
  uemb_rm = _tpose_call(user_emb.T)
  u = _sc_gather(users, uemb_rm)
  iemb_rm = _tpose_call(item_emb.T)
  v = _sc_gather(items, iemb_rm)
  w1t = W1.T
  w1u = w1t[:_EMB]
  w1v = w1t[_EMB:]
  out = _mlp_call(u, v, w1u, w1v, b1.reshape(1, -1), W2.T,
                  b2.reshape(1, -1), W3.T, b3.reshape(1, 1))
  return out.reshape(_BATCH)
